# baseline (speedup 1.0000x reference)
import functools

import jax
import jax.numpy as jnp
from jax import lax
from jax.experimental import pallas as pl
from jax.experimental.pallas import tpu as pltpu
from jax.experimental.pallas import tpu_sc as plsc

N = 10000
E = 320000
D = 128
H = 128
G = 16

NC = 2
NS = 16
NW = NC * NS

CHUNK = 96
EPW = E // NW
NFULL = EPW // CHUNK
TAIL = EPW - NFULL * CHUNK
NBUF = 3
PH = NFULL // 2
NR = PH // NBUF
PLEFT = PH - NR * NBUF
SHALF0 = PH * CHUNK
SHALF1 = EPW - SHALF0
ZCHUNK = 80
RCHUNKS = N // ZCHUNK

_SC_MESH = plsc.VectorSubcoreMesh(
    core_axis_name="c", subcore_axis_name="s", num_cores=NC, num_subcores=NS)


def _sc_degree(dstm, dstt):
    @functools.partial(
        pl.kernel,
        out_type=jax.ShapeDtypeStruct((NC * N,), jnp.float32),
        mesh=_SC_MESH,
        scratch_types=[
            pltpu.VMEM((NFULL, CHUNK), jnp.int32),
            pltpu.VMEM((TAIL,), jnp.int32),
            pltpu.VMEM((CHUNK,), jnp.float32),
            pltpu.VMEM((ZCHUNK,), jnp.float32),
            pltpu.VMEM_SHARED((N,), jnp.float32),
            pltpu.SemaphoreType.DMA,
        ],
    )
    def deg_kernel(dstm_hbm, dstt_hbm, out_hbm, dstidx, dstT, valbuf, zbuf,
                   acc, sem):
        c = lax.axis_index("c")
        s = lax.axis_index("s")
        wid = c * NS + s

        ones16 = jnp.ones((16,), jnp.float32)
        zero16 = jnp.zeros((16,), jnp.float32)

        def fill(i, _):
            valbuf[pl.ds(i * 16, 16)] = ones16
            return 0
        lax.fori_loop(0, CHUNK // 16, fill, 0)

        def zfill(i, _):
            zbuf[pl.ds(i * 16, 16)] = zero16
            return 0
        lax.fori_loop(0, ZCHUNK // 16, zfill, 0)

        pltpu.sync_copy(dstm_hbm.at[wid], dstidx)
        pltpu.sync_copy(dstt_hbm.at[wid], dstT)

        def acc_zero(j, _):
            k = s * 8 + j

            @pl.when(k < RCHUNKS)
            def _():
                pltpu.sync_copy(zbuf, acc.at[pl.ds(k * ZCHUNK, ZCHUNK)])
            return 0
        lax.fori_loop(0, 8, acc_zero, 0)
        plsc.subcore_barrier()

        def fire(g, _):
            pltpu.async_copy(valbuf, acc.at[dstidx.at[g]], sem, add=True)
            return 0
        lax.fori_loop(0, NFULL, fire, 0)

        def drain(g, _):
            pltpu.make_async_copy(valbuf, acc.at[dstidx.at[0]], sem).wait()
            return 0
        lax.fori_loop(0, NFULL, drain, 0)

        pltpu.sync_copy(valbuf.at[pl.ds(0, TAIL)], acc.at[dstT], add=True)
        plsc.subcore_barrier()

        obase = c * N

        def dump(j, _):
            k = s * 8 + j

            @pl.when(k < RCHUNKS)
            def _():
                pltpu.sync_copy(acc.at[pl.ds(k * ZCHUNK, ZCHUNK)], zbuf)
                pltpu.sync_copy(zbuf, out_hbm.at[pl.ds(obase + k * ZCHUNK, ZCHUNK)])
            return 0
        lax.fori_loop(0, 8, dump, 0)

    return deg_kernel(dstm, dstt)


def _sc_scatter(table, srcm, dstmA, dstmB, dstt):
    @functools.partial(
        pl.kernel,
        out_type=jax.ShapeDtypeStruct((NC, N, H), jnp.float32),
        mesh=_SC_MESH,
        scratch_types=[
            pltpu.VMEM((SHALF1,), jnp.int32),
            pltpu.VMEM((PH, CHUNK), jnp.int32),
            pltpu.VMEM((TAIL,), jnp.int32),
            [pltpu.VMEM((CHUNK, H), jnp.float32)] * NBUF,
            pltpu.VMEM_SHARED((N, H), jnp.float32),
            [pltpu.SemaphoreType.DMA] * NBUF,
            [pltpu.SemaphoreType.DMA] * NBUF,
        ],
    )
    def scat_kernel(table_hbm, srcm_hbm, dstmA_hbm, dstmB_hbm, dstt_hbm,
                    out_hbm, srcidx, dstidx, dstT,
                    rows, acc, gsems, ssems):
        c = lax.axis_index("c")
        s = lax.axis_index("s")
        wid = c * NS + s

        zero16 = jnp.zeros((16,), jnp.float32)

        pltpu.sync_copy(srcm_hbm.at[pl.ds(wid * EPW, SHALF0)],
                        srcidx.at[pl.ds(0, SHALF0)])
        pltpu.sync_copy(dstmA_hbm.at[wid], dstidx)
        pltpu.sync_copy(dstt_hbm.at[wid], dstT)
        pltpu.sync_copy(dstt_hbm.at[wid], dstT)

        def zrow(r, _):
            def zcol(cc, _):
                rows[0][r, pl.ds(cc * 16, 16)] = zero16
                return 0
            lax.fori_loop(0, H // 16, zcol, 0)
            return 0
        lax.fori_loop(0, ZCHUNK, zrow, 0)

        def acc_zero(j, _):
            k = s * 8 + j

            @pl.when(k < RCHUNKS)
            def _():
                pltpu.sync_copy(rows[0].at[pl.ds(0, ZCHUNK), :],
                                acc.at[pl.ds(k * ZCHUNK, ZCHUNK), :])
            return 0
        lax.fori_loop(0, 8, acc_zero, 0)
        plsc.subcore_barrier()

        def gather(off, b):
            pltpu.async_copy(table_hbm.at[srcidx.at[pl.ds(off, CHUNK)]],
                             rows[b], gsems[b])

        def gather_wait(b):
            pltpu.make_async_copy(
                table_hbm.at[srcidx.at[pl.ds(0, CHUNK)]], rows[b],
                gsems[b]).wait()

        def scatter(g, b):
            pltpu.async_copy(rows[b], acc.at[dstidx.at[g]], ssems[b],
                             add=True)

        def scatter_wait(b):
            pltpu.make_async_copy(rows[b], acc.at[dstidx.at[0]],
                                  ssems[b]).wait()


        for b in range(NBUF):
            gather(b * CHUNK, b)

        def round_body(r, _):
            for b in range(NBUF):
                gather_wait(b)
                scatter(NBUF * r + b, b)
            for b in range(NBUF):
                scatter_wait(b)
                l = NBUF * r + NBUF + b

                @pl.when(l < PH)
                def _():
                    gather(l * CHUNK, b)
            return 0
        lax.fori_loop(0, NR, round_body, 0)

        gather_wait(0)
        scatter(PH - 1, 0)
        scatter_wait(0)

        pltpu.sync_copy(srcm_hbm.at[pl.ds(wid * EPW + SHALF0, SHALF1)],
                        srcidx)
        pltpu.sync_copy(dstmB_hbm.at[wid], dstidx)

        for b in range(NBUF):
            gather(b * CHUNK, b)
        lax.fori_loop(0, NR, round_body, 0)
        gather_wait(0)
        scatter(PH - 1, 0)
        scatter_wait(0)

        pltpu.sync_copy(
            table_hbm.at[srcidx.at[pl.ds(PH * CHUNK, TAIL)]],
            rows[1].at[pl.ds(0, TAIL), :])
        pltpu.sync_copy(rows[1].at[pl.ds(0, TAIL), :], acc.at[dstT], add=True)
        plsc.subcore_barrier()

        def dump(j, _):
            k = s * 8 + j

            @pl.when(k < RCHUNKS)
            def _():
                pltpu.sync_copy(acc.at[pl.ds(k * ZCHUNK, ZCHUNK), :],
                                rows[0].at[pl.ds(0, ZCHUNK), :])
                pltpu.sync_copy(rows[0].at[pl.ds(0, ZCHUNK), :],
                                out_hbm.at[c, pl.ds(k * ZCHUNK, ZCHUNK), :])
            return 0
        lax.fori_loop(0, 8, dump, 0)

    return scat_kernel(table, srcm, dstmA, dstmB, dstt)


_BLK = 1000
_NBLK = N // _BLK


def _tc_prescale(x, W1, degp):
    def body(x_ref, w_ref, dp_ref, hw_ref, dinv_ref):
        deg = 1.0 + dp_ref[0] + dp_ref[1]
        dinv = lax.rsqrt(deg)
        dinv_ref[...] = dinv
        hw_ref[...] = jnp.dot(x_ref[...], w_ref[...],
                              preferred_element_type=jnp.float32) * dinv

    return pl.pallas_call(
        body,
        grid=(_NBLK,),
        in_specs=[
            pl.BlockSpec((_BLK, D), lambda i: (i, 0)),
            pl.BlockSpec((D, H), lambda i: (0, 0)),
            pl.BlockSpec((NC, _BLK, 1), lambda i: (0, i, 0)),
        ],
        out_specs=[
            pl.BlockSpec((_BLK, H), lambda i: (i, 0)),
            pl.BlockSpec((_BLK, 1), lambda i: (i, 0)),
        ],
        out_shape=[
            jax.ShapeDtypeStruct((N, H), jnp.float32),
            jax.ShapeDtypeStruct((N, 1), jnp.float32),
        ],
    )(x, W1, degp)


def _tc_layer_mid(Sp, hwp, dinv, b, W2):
    def body(s_ref, hw_ref, dinv_ref, b_ref, w_ref, out_ref):
        dinv = dinv_ref[...]
        h = s_ref[0] + s_ref[1] + hw_ref[...]
        h = jnp.maximum(dinv * h + b_ref[...], 0.0)
        out_ref[...] = jnp.dot(h, w_ref[...],
                               preferred_element_type=jnp.float32) * dinv

    return pl.pallas_call(
        body,
        grid=(_NBLK,),
        in_specs=[
            pl.BlockSpec((NC, _BLK, H), lambda i: (0, i, 0)),
            pl.BlockSpec((_BLK, H), lambda i: (i, 0)),
            pl.BlockSpec((_BLK, 1), lambda i: (i, 0)),
            pl.BlockSpec((1, H), lambda i: (0, 0)),
            pl.BlockSpec((H, H), lambda i: (0, 0)),
        ],
        out_specs=pl.BlockSpec((_BLK, H), lambda i: (i, 0)),
        out_shape=jax.ShapeDtypeStruct((N, H), jnp.float32),
    )(Sp, hwp, dinv, b, W2)


def _tc_finish_pool(Sp, hwp, dinv, b, batch2d):
    def body(s_ref, hw_ref, dinv_ref, b_ref, bat_ref, out_ref, cnt_ref):
        i = pl.program_id(0)
        dinv = dinv_ref[...]
        h = s_ref[0] + s_ref[1] + hw_ref[...]
        h = jnp.maximum(dinv * h + b_ref[...], 0.0)

        gids = lax.broadcasted_iota(jnp.int32, (_BLK, G), 1)
        onehot = (bat_ref[...] == gids).astype(jnp.float32)
        part = lax.dot_general(onehot, h, (((0,), (0,)), ((), ())),
                               preferred_element_type=jnp.float32)
        pcnt = lax.dot_general(onehot, jnp.ones((_BLK, 1), jnp.float32),
                               (((0,), (0,)), ((), ())),
                               preferred_element_type=jnp.float32)

        @pl.when(i == 0)
        def _():
            out_ref[...] = jnp.zeros_like(out_ref)
            cnt_ref[...] = jnp.zeros_like(cnt_ref)

        out_ref[...] += part
        cnt_ref[...] += pcnt

        @pl.when(i == _NBLK - 1)
        def _():
            out_ref[...] = out_ref[...] / jnp.maximum(cnt_ref[...], 1.0)

    return pl.pallas_call(
        body,
        grid=(_NBLK,),
        in_specs=[
            pl.BlockSpec((NC, _BLK, H), lambda i: (0, i, 0)),
            pl.BlockSpec((_BLK, H), lambda i: (i, 0)),
            pl.BlockSpec((_BLK, 1), lambda i: (i, 0)),
            pl.BlockSpec((1, H), lambda i: (0, 0)),
            pl.BlockSpec((_BLK, 1), lambda i: (i, 0)),
        ],
        out_specs=pl.BlockSpec((G, H), lambda i: (0, 0)),
        out_shape=jax.ShapeDtypeStruct((G, H), jnp.float32),
        scratch_shapes=[pltpu.VMEM((G, 1), jnp.float32)],
    )(Sp, hwp, dinv, b, batch2d)


def kernel(x, edge_index, batch, W1, b1, W2, b2):
    srcf = edge_index[0]
    dst2 = edge_index[1].reshape(NW, EPW)
    dstm = dst2[:, :NFULL * CHUNK].reshape(NW, NFULL, CHUNK)
    dstt = dst2[:, NFULL * CHUNK:]
    dstmA = dstm[:, :PH, :]
    dstmB = dstm[:, PH:, :]

    degp = _sc_degree(dstm, dstt)
    degp3 = degp.reshape(NC, N, 1)

    hw1p, dinv = _tc_prescale(x, W1, degp3)
    S1 = _sc_scatter(hw1p, srcf, dstmA, dstmB, dstt)
    hw2p = _tc_layer_mid(S1, hw1p, dinv, b1.reshape(1, H), W2)
    S2 = _sc_scatter(hw2p, srcf, dstmA, dstmB, dstt)
    g = _tc_finish_pool(S2, hw2p, dinv, b2.reshape(1, H),
                        batch.reshape(N, 1))
    return g
